# baseline (speedup 1.0000x reference)
import jax
import jax.numpy as jnp
from jax import lax
from jax.experimental import pallas as pl
from jax.experimental.pallas import tpu as pltpu
from jax.experimental.pallas import tpu_sc as plsc

N = 10000
E = 320000
R = 8
H = 128

NC = 2
NS = 16
PPC = 10
NPART = NC * PPC
NCHUNK = 512
ACC_ROWS = R * NCHUNK
TRASH = ACC_ROWS
ZROWS = 264
ACC_TOTAL = NS * ZROWS
WHALF = NCHUNK // 2
EW = E // NS
EB = 2000
K = 128
KSH = 7
NBMAX = (EW + 2 * K - 1) // K
NBMAX = ((NBMAX + 15) // 16) * 16
CB_ROWS = 48
CLIVE = ACC_ROWS // H

_i32 = jnp.int32
_f32 = jnp.float32


def _make_prep():
    out_type = (
        jax.ShapeDtypeStruct((NC, NS, PPC, NBMAX, K), _i32),
        jax.ShapeDtypeStruct((NC, NS, PPC, NBMAX, K), _i32),
        jax.ShapeDtypeStruct((NC, NS, PPC, 16), _i32),
        jax.ShapeDtypeStruct((NPART, CLIVE, H), _f32),
    )

    scratch = [
        pltpu.VMEM((NBMAX, K), _i32),
        pltpu.VMEM((NBMAX, K), _i32),
        pltpu.VMEM((EB,), _i32),
        pltpu.VMEM((EB,), _i32),
        pltpu.VMEM((EB,), _i32),
        pltpu.VMEM((16,), _i32),
        pltpu.VMEM((CB_ROWS, H), _f32),
        pltpu.VMEM((CB_ROWS, H), _f32),
        pltpu.VMEM((CB_ROWS,), _i32),
        pltpu.SemaphoreType.DMA,
        pltpu.VMEM_SHARED((CB_ROWS, H), _f32),
    ]

    def body(src_hbm, dst_hbm, rel_hbm, glist, flist, nbv, c_hbm,
             gbuf, fbuf, ebuf_s, ebuf_d, ebuf_r, nstage, cb, zbuf48,
             idrows, sem, cnt_sh):
        cid = lax.axis_index("c")
        sid = lax.axis_index("s")

        zero16f = jnp.zeros((16,), _f32)
        ones16f = jnp.ones((16,), _f32)
        lane = lax.iota(_i32, 16)

        def zb(i, _):
            rr = i >> 3
            cc = (i & 7) * 16
            zbuf48[rr, pl.ds(cc, 16)] = zero16f
            return 0
        lax.fori_loop(0, CB_ROWS * 8, zb, 0)
        for j in range(CB_ROWS // 16):
            idrows[pl.ds(j * 16, 16)] = lane + j * 16

        def partition(pp, _):
            lo = (cid * PPC + pp) * NCHUNK

            def zcb(i, _):
                rr = i >> 3
                cc = (i & 7) * 16
                cb[rr, pl.ds(cc, 16)] = zero16f
                return 0
            lax.fori_loop(0, CB_ROWS * 8, zcb, 0)

            @pl.when(sid == 0)
            def _():
                pltpu.sync_copy(zbuf48, cnt_sh)
            plsc.subcore_barrier()

            def scan_block(blk, off_v):
                ebase = sid * EW + blk * EB
                c1 = pltpu.async_copy(src_hbm.at[pl.ds(ebase, EB)],
                                      ebuf_s, sem)
                c2 = pltpu.async_copy(dst_hbm.at[pl.ds(ebase, EB)],
                                      ebuf_d, sem)
                c3 = pltpu.async_copy(rel_hbm.at[pl.ds(ebase, EB)],
                                      ebuf_r, sem)
                c1.wait(); c2.wait(); c3.wait()

                def inner(i, off_v):
                    d = ebuf_d[pl.ds(i * 16, 16)]
                    s = ebuf_s[pl.ds(i * 16, 16)]
                    r = ebuf_r[pl.ds(i * 16, 16)]
                    m = (d >= lo) & (d < lo + NCHUNK)
                    si = r * NCHUNK + (d - lo)
                    srow = lax.shift_right_logical(si, 7)
                    scol = lax.bitwise_and(si, H - 1)
                    plsc.addupdate_scatter(cb, [srow, scol], ones16f,
                                           mask=m)
                    ones = jnp.where(m, 1, 0)
                    pos = off_v + plsc.cumsum(ones) - 1
                    prow = lax.shift_right_logical(pos, KSH)
                    pcol = lax.bitwise_and(pos, K - 1)
                    plsc.store_scatter(gbuf, [prow, pcol], s, mask=m)
                    plsc.store_scatter(fbuf, [prow, pcol], si, mask=m)
                    return off_v + plsc.all_reduce_population_count(m)

                return lax.fori_loop(0, EB // 16, inner, off_v)

            off_v = lax.fori_loop(0, EW // EB, scan_block,
                                  jnp.zeros((16,), _i32))
            matched = jnp.sum(jnp.where(lane == 0, off_v, 0))

            trash16 = jnp.full((16,), TRASH, _i32)
            zero16i = jnp.zeros((16,), _i32)
            def padw(j, _):
                p = matched + j * 16 + lane
                prow = lax.shift_right_logical(p, KSH)
                pcol = lax.bitwise_and(p, K - 1)
                plsc.store_scatter(fbuf, [prow, pcol], trash16)
                plsc.store_scatter(gbuf, [prow, pcol], zero16i)
                return 0
            lax.fori_loop(0, K // 16, padw, 0)

            nb = (matched + (K - 1)) // K
            nstage[pl.ds(0, 16)] = jnp.zeros((16,), _i32) + nb
            pltpu.sync_copy(nstage, nbv.at[cid, sid, pp])

            nch = (matched + K + 16 * K - 1) // (16 * K)
            def wch(j, _):
                w1 = pltpu.async_copy(
                    gbuf.at[pl.ds(j * 16, 16)],
                    glist.at[cid, sid, pp, pl.ds(j * 16, 16)], sem)
                w2 = pltpu.async_copy(
                    fbuf.at[pl.ds(j * 16, 16)],
                    flist.at[cid, sid, pp, pl.ds(j * 16, 16)], sem)
                w1.wait(); w2.wait()
                return 0
            lax.fori_loop(0, nch, wch, 0)

            pltpu.sync_copy(cb, cnt_sh.at[idrows], add=True)
            plsc.subcore_barrier()

            @pl.when(sid == 0)
            def _():
                pltpu.sync_copy(cnt_sh.at[pl.ds(0, CLIVE)],
                                c_hbm.at[cid * PPC + pp])
            plsc.subcore_barrier()
            return 0

        lax.fori_loop(0, PPC, partition, 0)

    mesh = plsc.VectorSubcoreMesh(core_axis_name="c", subcore_axis_name="s")
    return pl.kernel(body, out_type=out_type,
                     mesh=mesh, scratch_types=scratch,
                     compiler_params=pltpu.CompilerParams(
                         needs_layout_passes=False))


def _make_agg():
    out_type = jax.ShapeDtypeStruct((R, N, H), _f32)

    scratch = [
        pltpu.VMEM((NBMAX, K), _i32),
        pltpu.VMEM((NBMAX, K), _i32),
        pltpu.VMEM((16,), _i32),
        pltpu.VMEM((K, H), _f32),
        pltpu.VMEM((K, H), _f32),
        pltpu.VMEM((K, H), _f32),
        pltpu.VMEM((32, H), _f32),
        pltpu.SemaphoreType.DMA,
        pltpu.SemaphoreType.DMA,
        pltpu.SemaphoreType.DMA,
        pltpu.VMEM_SHARED((ACC_TOTAL, H), _f32),
    ]

    def body(x_hbm, glist, flist, nbv, a_hbm,
             gbuf, fbuf, nstage, rows0, rows1, rows2, zbuf,
             gsem, ssem, lsem, acc_sh):
        cid = lax.axis_index("c")
        sid = lax.axis_index("s")
        rows = (rows0, rows1, rows2)

        zero16f = jnp.zeros((16,), _f32)
        lane = lax.iota(_i32, 16)

        def zb(i, _):
            rr = i >> 3
            cc = (i & 7) * 16
            zbuf[rr, pl.ds(cc, 16)] = zero16f
            return 0
        lax.fori_loop(0, 32 * 8, zb, 0)

        def partition(pp, _):
            lo = (cid * PPC + pp) * NCHUNK

            pltpu.sync_copy(nbv.at[cid, sid, pp], nstage)
            nv = nstage[pl.ds(0, 16)]
            nb = jnp.sum(jnp.where(lane == 0, nv, 0))
            nch = (nb + 15) // 16
            def rch(j, _):
                r1 = pltpu.async_copy(
                    glist.at[cid, sid, pp, pl.ds(j * 16, 16)],
                    gbuf.at[pl.ds(j * 16, 16)], lsem)
                r2 = pltpu.async_copy(
                    flist.at[cid, sid, pp, pl.ds(j * 16, 16)],
                    fbuf.at[pl.ds(j * 16, 16)], lsem)
                r1.wait(); r2.wait()
                return 0
            lax.fori_loop(0, nch, rch, 0)

            base = pl.multiple_of(sid * ZROWS, 8)
            for j in range(ZROWS // 32):
                pltpu.sync_copy(zbuf, acc_sh.at[pl.ds(base + j * 32, 32)])
            pltpu.sync_copy(zbuf.at[pl.ds(0, ZROWS % 32)],
                            acc_sh.at[pl.ds(base + (ZROWS // 32) * 32,
                                            ZROWS % 32)])

            plsc.subcore_barrier()

            for q in range(3):
                @pl.when(q < nb)
                def _(q=q):
                    pltpu.async_copy(x_hbm.at[gbuf.at[q]], rows[q], gsem)

            ntri = (nb + 2) // 3
            def tri(g, _):
                b = 3 * g
                for q in range(3):
                    @pl.when(b + q < nb)
                    def _(q=q):
                        pltpu.make_async_copy(x_hbm.at[gbuf.at[b + q]],
                                              rows[q], gsem).wait()
                        pltpu.sync_copy(rows[q], acc_sh.at[fbuf.at[b + q]],
                                        add=True)

                    @pl.when(b + 3 + q < nb)
                    def _(q=q):
                        pltpu.async_copy(x_hbm.at[gbuf.at[b + 3 + q]],
                                         rows[q], gsem)
                return 0
            lax.fori_loop(0, ntri, tri, 0)

            plsc.subcore_barrier()

            r = sid % R
            half = sid // R
            row0 = pl.multiple_of(r * NCHUNK + half * WHALF, 8)
            node0 = pl.multiple_of(lo + half * WHALF, 8)

            @pl.when(node0 + WHALF <= N)
            def _():
                pltpu.sync_copy(acc_sh.at[pl.ds(row0, WHALF)],
                                a_hbm.at[r, pl.ds(node0, WHALF)])

            tail = N % WHALF

            @pl.when((node0 + WHALF > N) & (node0 < N))
            def _():
                pltpu.sync_copy(acc_sh.at[pl.ds(row0, tail)],
                                a_hbm.at[r, pl.ds(node0, tail)])
            plsc.subcore_barrier()
            return 0

        lax.fori_loop(0, PPC, partition, 0)

    mesh = plsc.VectorSubcoreMesh(core_axis_name="c", subcore_axis_name="s")
    return pl.kernel(body, out_type=out_type,
                     mesh=mesh, scratch_types=scratch,
                     compiler_params=pltpu.CompilerParams(
                         needs_layout_passes=False))


_prep = _make_prep()
_agg = _make_agg()


_BM = 512
_BOUNDS = (0, 4000, 7000, 9000, 9500, 10000)


def _proj_body(x_ref, w_ref, b_ref, o_ref):
    m = pl.program_id(0)
    row = m * _BM + lax.broadcasted_iota(_i32, (_BM, 1), 0)
    x = x_ref[...]
    acc = jnp.zeros((_BM, H), _f32)
    for t in range(5):
        y = jnp.dot(x, w_ref[t], preferred_element_type=_f32) + b_ref[t]
        sel = (row >= _BOUNDS[t]) & (row < _BOUNDS[t + 1])
        acc = jnp.where(sel, y, acc)
    o_ref[...] = acc


def _proj(x_cat, w_stack, b_stack):
    grid = (pl.cdiv(N, _BM),)
    return pl.pallas_call(
        _proj_body,
        grid=grid,
        in_specs=[
            pl.BlockSpec((_BM, H), lambda m: (m, 0)),
            pl.BlockSpec((5, H, H), lambda m: (0, 0, 0)),
            pl.BlockSpec((5, H), lambda m: (0, 0)),
        ],
        out_specs=pl.BlockSpec((_BM, H), lambda m: (m, 0)),
        out_shape=jax.ShapeDtypeStruct((N, H), _f32),
    )(x_cat, w_stack, b_stack)


def _make_conv_tc(relu):
    def body(x_ref, a_ref, c_ref, wroot_ref, wrel_ref, b_ref, o_ref):
        acc = jnp.dot(x_ref[...], wroot_ref[...], preferred_element_type=_f32)
        for r in range(R):
            cnt = c_ref[r][:, None]
            rec = 1.0 / jnp.maximum(cnt, 1.0)
            acc = acc + jnp.dot(a_ref[r] * rec, wrel_ref[r],
                                preferred_element_type=_f32)
        acc = acc + b_ref[...]
        if relu:
            acc = jnp.maximum(acc, 0.0)
        o_ref[...] = acc

    def run(x, a, cnt, w_root, w_rel, b):
        grid = (pl.cdiv(N, _BM),)
        return pl.pallas_call(
            body,
            grid=grid,
            in_specs=[
                pl.BlockSpec((_BM, H), lambda m: (m, 0)),
                pl.BlockSpec((R, _BM, H), lambda m: (0, m, 0)),
                pl.BlockSpec((R, _BM), lambda m: (0, m)),
                pl.BlockSpec((H, H), lambda m: (0, 0)),
                pl.BlockSpec((R, H, H), lambda m: (0, 0, 0)),
                pl.BlockSpec((H,), lambda m: (0,)),
            ],
            out_specs=pl.BlockSpec((_BM, H), lambda m: (m, 0)),
            out_shape=jax.ShapeDtypeStruct((N, H), _f32),
        )(x, a, cnt, w_root, w_rel, b)

    return run


_conv_relu = _make_conv_tc(True)
_conv_plain = _make_conv_tc(False)


def kernel(x_user, x_food, x_ingredient, x_category, x_habit, edge_index,
           edge_type, W_user, b_user, W_food, b_food, W_ing, b_ing, W_cat,
           b_cat, W_hab, b_hab, W_rel1, W_root1, b1, W_rel2, W_root2, b2):
    x_cat = jnp.concatenate([x_user, x_food, x_ingredient, x_category,
                             x_habit], axis=0).astype(_f32)
    w_stack = jnp.stack([W_user, W_food, W_ing, W_cat, W_hab])
    b_stack = jnp.stack([b_user, b_food, b_ing, b_cat, b_hab])
    x_all = _proj(x_cat, w_stack, b_stack)

    src = edge_index[0].astype(_i32)
    dst = edge_index[1].astype(_i32)
    et = edge_type.astype(_i32)

    glist, flist, nbv, craw = _prep(src, dst, et)
    cnt = (craw.reshape(NPART, R, NCHUNK)
               .transpose(1, 0, 2).reshape(R, NPART * NCHUNK)[:, :N])

    a1 = _agg(x_all, glist, flist, nbv)
    h = _conv_relu(x_all, a1, cnt, W_root1, W_rel1, b1)
    a2 = _agg(h, glist, flist, nbv)
    out = _conv_plain(h, a2, cnt, W_root2, W_rel2, b2)
    return out


# speedup vs baseline: 1.1706x; 1.1706x over previous
import jax
import jax.numpy as jnp
from jax import lax
from jax.experimental import pallas as pl
from jax.experimental.pallas import tpu as pltpu
from jax.experimental.pallas import tpu_sc as plsc

N = 10000
E = 320000
R = 8
H = 128

NC = 2
NS = 16
PPC = 8
NPART = NC * PPC
NCHUNK = 640
ACC_ROWS = R * NCHUNK
TRASH = ACC_ROWS
ZROWS = 328
ACC_TOTAL = NS * ZROWS
WHALF = NCHUNK // 2
EW = E // NS
EB = 2000
K = 128
KSH = 7
NBMAX = (EW + 2 * K - 1) // K
NBMAX = ((NBMAX + 15) // 16) * 16
CB_ROWS = 48
CLIVE = ACC_ROWS // H

_i32 = jnp.int32
_f32 = jnp.float32


def _make_prep():
    out_type = (
        jax.ShapeDtypeStruct((NC, NS, PPC, NBMAX, K), _i32),
        jax.ShapeDtypeStruct((NC, NS, PPC, NBMAX, K), _i32),
        jax.ShapeDtypeStruct((NC, NS, PPC, 16), _i32),
        jax.ShapeDtypeStruct((NPART, CLIVE, H), _f32),
    )

    scratch = [
        pltpu.VMEM((NBMAX, K), _i32),
        pltpu.VMEM((NBMAX, K), _i32),
        pltpu.VMEM((EB,), _i32),
        pltpu.VMEM((EB,), _i32),
        pltpu.VMEM((EB,), _i32),
        pltpu.VMEM((16,), _i32),
        pltpu.VMEM((CB_ROWS, H), _f32),
        pltpu.VMEM((CB_ROWS, H), _f32),
        pltpu.VMEM((CB_ROWS,), _i32),
        pltpu.SemaphoreType.DMA,
        pltpu.VMEM_SHARED((CB_ROWS, H), _f32),
    ]

    def body(src_hbm, dst_hbm, rel_hbm, glist, flist, nbv, c_hbm,
             gbuf, fbuf, ebuf_s, ebuf_d, ebuf_r, nstage, cb, zbuf48,
             idrows, sem, cnt_sh):
        cid = lax.axis_index("c")
        sid = lax.axis_index("s")

        zero16f = jnp.zeros((16,), _f32)
        ones16f = jnp.ones((16,), _f32)
        lane = lax.iota(_i32, 16)

        def zb(i, _):
            rr = i >> 3
            cc = (i & 7) * 16
            zbuf48[rr, pl.ds(cc, 16)] = zero16f
            return 0
        lax.fori_loop(0, CB_ROWS * 8, zb, 0)
        for j in range(CB_ROWS // 16):
            idrows[pl.ds(j * 16, 16)] = lane + j * 16

        def partition(pp, _):
            lo = (cid * PPC + pp) * NCHUNK

            def zcb(i, _):
                rr = i >> 3
                cc = (i & 7) * 16
                cb[rr, pl.ds(cc, 16)] = zero16f
                return 0
            lax.fori_loop(0, CB_ROWS * 8, zcb, 0)

            @pl.when(sid == 0)
            def _():
                pltpu.sync_copy(zbuf48, cnt_sh)
            plsc.subcore_barrier()

            def scan_block(blk, off_v):
                ebase = sid * EW + blk * EB
                c1 = pltpu.async_copy(src_hbm.at[pl.ds(ebase, EB)],
                                      ebuf_s, sem)
                c2 = pltpu.async_copy(dst_hbm.at[pl.ds(ebase, EB)],
                                      ebuf_d, sem)
                c3 = pltpu.async_copy(rel_hbm.at[pl.ds(ebase, EB)],
                                      ebuf_r, sem)
                c1.wait(); c2.wait(); c3.wait()

                def inner(i, off_v):
                    d = ebuf_d[pl.ds(i * 16, 16)]
                    s = ebuf_s[pl.ds(i * 16, 16)]
                    r = ebuf_r[pl.ds(i * 16, 16)]
                    m = (d >= lo) & (d < lo + NCHUNK)
                    si = r * NCHUNK + (d - lo)
                    srow = lax.shift_right_logical(si, 7)
                    scol = lax.bitwise_and(si, H - 1)
                    plsc.addupdate_scatter(cb, [srow, scol], ones16f,
                                           mask=m)
                    ones = jnp.where(m, 1, 0)
                    pos = off_v + plsc.cumsum(ones) - 1
                    prow = lax.shift_right_logical(pos, KSH)
                    pcol = lax.bitwise_and(pos, K - 1)
                    plsc.store_scatter(gbuf, [prow, pcol], s, mask=m)
                    plsc.store_scatter(fbuf, [prow, pcol], si, mask=m)
                    return off_v + plsc.all_reduce_population_count(m)

                return lax.fori_loop(0, EB // 16, inner, off_v)

            off_v = lax.fori_loop(0, EW // EB, scan_block,
                                  jnp.zeros((16,), _i32))
            matched = jnp.sum(jnp.where(lane == 0, off_v, 0))

            trash16 = jnp.full((16,), TRASH, _i32)
            zero16i = jnp.zeros((16,), _i32)
            def padw(j, _):
                p = matched + j * 16 + lane
                prow = lax.shift_right_logical(p, KSH)
                pcol = lax.bitwise_and(p, K - 1)
                plsc.store_scatter(fbuf, [prow, pcol], trash16)
                plsc.store_scatter(gbuf, [prow, pcol], zero16i)
                return 0
            lax.fori_loop(0, K // 16, padw, 0)

            nb = (matched + (K - 1)) // K
            nstage[pl.ds(0, 16)] = jnp.zeros((16,), _i32) + nb
            pltpu.sync_copy(nstage, nbv.at[cid, sid, pp])

            nch = (matched + K + 16 * K - 1) // (16 * K)
            def wch(j, _):
                w1 = pltpu.async_copy(
                    gbuf.at[pl.ds(j * 16, 16)],
                    glist.at[cid, sid, pp, pl.ds(j * 16, 16)], sem)
                w2 = pltpu.async_copy(
                    fbuf.at[pl.ds(j * 16, 16)],
                    flist.at[cid, sid, pp, pl.ds(j * 16, 16)], sem)
                w1.wait(); w2.wait()
                return 0
            lax.fori_loop(0, nch, wch, 0)

            pltpu.sync_copy(cb, cnt_sh.at[idrows], add=True)
            plsc.subcore_barrier()

            @pl.when(sid == 0)
            def _():
                pltpu.sync_copy(cnt_sh.at[pl.ds(0, CLIVE)],
                                c_hbm.at[cid * PPC + pp])
            plsc.subcore_barrier()
            return 0

        lax.fori_loop(0, PPC, partition, 0)

    mesh = plsc.VectorSubcoreMesh(core_axis_name="c", subcore_axis_name="s")
    return pl.kernel(body, out_type=out_type,
                     mesh=mesh, scratch_types=scratch,
                     compiler_params=pltpu.CompilerParams(
                         needs_layout_passes=False))


def _make_agg():
    out_type = jax.ShapeDtypeStruct((R, N, H), _f32)

    scratch = [
        pltpu.VMEM((NBMAX, K), _i32),
        pltpu.VMEM((NBMAX, K), _i32),
        pltpu.VMEM((16,), _i32),
        pltpu.VMEM((K, H), _f32),
        pltpu.VMEM((K, H), _f32),
        pltpu.VMEM((64, H), _f32),
        pltpu.SemaphoreType.DMA,
        pltpu.SemaphoreType.DMA,
        pltpu.SemaphoreType.DMA,
        pltpu.VMEM_SHARED((ACC_TOTAL, H), _f32),
    ]

    def body(x_hbm, glist, flist, nbv, a_hbm,
             gbuf, fbuf, nstage, rows0, rows1, zbuf,
             gsem, ssem, lsem, acc_sh):
        cid = lax.axis_index("c")
        sid = lax.axis_index("s")

        zero16f = jnp.zeros((16,), _f32)
        lane = lax.iota(_i32, 16)

        def zb(i, _):
            rr = i >> 3
            cc = (i & 7) * 16
            zbuf[rr, pl.ds(cc, 16)] = zero16f
            return 0
        lax.fori_loop(0, 64 * 8, zb, 0)

        def partition(pp, _):
            lo = (cid * PPC + pp) * NCHUNK

            pltpu.sync_copy(nbv.at[cid, sid, pp], nstage)
            nv = nstage[pl.ds(0, 16)]
            nb = jnp.sum(jnp.where(lane == 0, nv, 0))
            nch = (nb + 15) // 16
            def rch(j, _):
                r1 = pltpu.async_copy(
                    glist.at[cid, sid, pp, pl.ds(j * 16, 16)],
                    gbuf.at[pl.ds(j * 16, 16)], lsem)
                r2 = pltpu.async_copy(
                    flist.at[cid, sid, pp, pl.ds(j * 16, 16)],
                    fbuf.at[pl.ds(j * 16, 16)], lsem)
                r1.wait(); r2.wait()
                return 0
            lax.fori_loop(0, nch, rch, 0)

            base = pl.multiple_of(sid * ZROWS, 8)
            for j in range(ZROWS // 64):
                pltpu.sync_copy(zbuf, acc_sh.at[pl.ds(base + j * 64, 64)])
            pltpu.sync_copy(zbuf.at[pl.ds(0, ZROWS % 64)],
                            acc_sh.at[pl.ds(base + (ZROWS // 64) * 64,
                                            ZROWS % 64)])

            plsc.subcore_barrier()

            @pl.when(nb > 0)
            def _():
                pltpu.async_copy(x_hbm.at[gbuf.at[0]], rows0, gsem)

            npairs = (nb + 1) // 2
            def pair(g, _):
                b0 = 2 * g
                b1 = b0 + 1
                pltpu.make_async_copy(x_hbm.at[gbuf.at[b0]], rows0,
                                      gsem).wait()

                @pl.when(b1 < nb)
                def _():
                    pltpu.async_copy(x_hbm.at[gbuf.at[b1]], rows1, ssem)
                pltpu.sync_copy(rows0, acc_sh.at[fbuf.at[b0]], add=True)

                @pl.when(b1 < nb)
                def _():
                    pltpu.make_async_copy(x_hbm.at[gbuf.at[b1]], rows1,
                                          ssem).wait()

                    @pl.when(b1 + 1 < nb)
                    def _():
                        pltpu.async_copy(x_hbm.at[gbuf.at[b1 + 1]], rows0,
                                         gsem)
                    pltpu.sync_copy(rows1, acc_sh.at[fbuf.at[b1]],
                                    add=True)
                return 0
            lax.fori_loop(0, npairs, pair, 0)

            plsc.subcore_barrier()

            r = sid % R
            half = sid // R
            row0 = pl.multiple_of(r * NCHUNK + half * WHALF, 8)
            node0 = pl.multiple_of(lo + half * WHALF, 8)

            @pl.when(node0 + WHALF <= N)
            def _():
                pltpu.sync_copy(acc_sh.at[pl.ds(row0, WHALF)],
                                a_hbm.at[r, pl.ds(node0, WHALF)])

            tail = N % WHALF

            @pl.when((node0 + WHALF > N) & (node0 < N))
            def _():
                pltpu.sync_copy(acc_sh.at[pl.ds(row0, tail)],
                                a_hbm.at[r, pl.ds(node0, tail)])
            plsc.subcore_barrier()
            return 0

        lax.fori_loop(0, PPC, partition, 0)

    mesh = plsc.VectorSubcoreMesh(core_axis_name="c", subcore_axis_name="s")
    return pl.kernel(body, out_type=out_type,
                     mesh=mesh, scratch_types=scratch,
                     compiler_params=pltpu.CompilerParams(
                         needs_layout_passes=False))


_prep = _make_prep()
_agg = _make_agg()


_BM = 512
_BOUNDS = (0, 4000, 7000, 9000, 9500, 10000)


def _proj_body(x_ref, w_ref, b_ref, o_ref):
    m = pl.program_id(0)
    row = m * _BM + lax.broadcasted_iota(_i32, (_BM, 1), 0)
    x = x_ref[...]
    acc = jnp.zeros((_BM, H), _f32)
    for t in range(5):
        y = jnp.dot(x, w_ref[t], preferred_element_type=_f32) + b_ref[t]
        sel = (row >= _BOUNDS[t]) & (row < _BOUNDS[t + 1])
        acc = jnp.where(sel, y, acc)
    o_ref[...] = acc


def _proj(x_cat, w_stack, b_stack):
    grid = (pl.cdiv(N, _BM),)
    return pl.pallas_call(
        _proj_body,
        grid=grid,
        in_specs=[
            pl.BlockSpec((_BM, H), lambda m: (m, 0)),
            pl.BlockSpec((5, H, H), lambda m: (0, 0, 0)),
            pl.BlockSpec((5, H), lambda m: (0, 0)),
        ],
        out_specs=pl.BlockSpec((_BM, H), lambda m: (m, 0)),
        out_shape=jax.ShapeDtypeStruct((N, H), _f32),
    )(x_cat, w_stack, b_stack)


def _make_conv_tc(relu):
    def body(x_ref, a_ref, c_ref, wroot_ref, wrel_ref, b_ref, o_ref):
        acc = jnp.dot(x_ref[...], wroot_ref[...], preferred_element_type=_f32)
        for r in range(R):
            cnt = c_ref[r][:, None]
            rec = 1.0 / jnp.maximum(cnt, 1.0)
            acc = acc + jnp.dot(a_ref[r] * rec, wrel_ref[r],
                                preferred_element_type=_f32)
        acc = acc + b_ref[...]
        if relu:
            acc = jnp.maximum(acc, 0.0)
        o_ref[...] = acc

    def run(x, a, cnt, w_root, w_rel, b):
        grid = (pl.cdiv(N, _BM),)
        return pl.pallas_call(
            body,
            grid=grid,
            in_specs=[
                pl.BlockSpec((_BM, H), lambda m: (m, 0)),
                pl.BlockSpec((R, _BM, H), lambda m: (0, m, 0)),
                pl.BlockSpec((R, _BM), lambda m: (0, m)),
                pl.BlockSpec((H, H), lambda m: (0, 0)),
                pl.BlockSpec((R, H, H), lambda m: (0, 0, 0)),
                pl.BlockSpec((H,), lambda m: (0,)),
            ],
            out_specs=pl.BlockSpec((_BM, H), lambda m: (m, 0)),
            out_shape=jax.ShapeDtypeStruct((N, H), _f32),
        )(x, a, cnt, w_root, w_rel, b)

    return run


_conv_relu = _make_conv_tc(True)
_conv_plain = _make_conv_tc(False)


def kernel(x_user, x_food, x_ingredient, x_category, x_habit, edge_index,
           edge_type, W_user, b_user, W_food, b_food, W_ing, b_ing, W_cat,
           b_cat, W_hab, b_hab, W_rel1, W_root1, b1, W_rel2, W_root2, b2):
    x_cat = jnp.concatenate([x_user, x_food, x_ingredient, x_category,
                             x_habit], axis=0).astype(_f32)
    w_stack = jnp.stack([W_user, W_food, W_ing, W_cat, W_hab])
    b_stack = jnp.stack([b_user, b_food, b_ing, b_cat, b_hab])
    x_all = _proj(x_cat, w_stack, b_stack)

    src = edge_index[0].astype(_i32)
    dst = edge_index[1].astype(_i32)
    et = edge_type.astype(_i32)

    glist, flist, nbv, craw = _prep(src, dst, et)
    cnt = (craw.reshape(NPART, R, NCHUNK)
               .transpose(1, 0, 2).reshape(R, NPART * NCHUNK)[:, :N])

    a1 = _agg(x_all, glist, flist, nbv)
    h = _conv_relu(x_all, a1, cnt, W_root1, W_rel1, b1)
    a2 = _agg(h, glist, flist, nbv)
    out = _conv_plain(h, a2, cnt, W_root2, W_rel2, b2)
    return out


# speedup vs baseline: 1.1814x; 1.0092x over previous
import jax
import jax.numpy as jnp
from jax import lax
from jax.experimental import pallas as pl
from jax.experimental.pallas import tpu as pltpu
from jax.experimental.pallas import tpu_sc as plsc

N = 10000
E = 320000
R = 8
H = 128

NC = 2
NS = 16
PPC = 8
NPART = NC * PPC
NCHUNK = 640
ACC_ROWS = R * NCHUNK
TRASH = ACC_ROWS
ZROWS = 328
ACC_TOTAL = NS * ZROWS
WHALF = NCHUNK // 2
EW = E // NS
EB = 2000
K = 128
KSH = 7
NBMAX = (EW + 2 * K - 1) // K
NBMAX = ((NBMAX + 15) // 16) * 16
CB_ROWS = 48
CLIVE = ACC_ROWS // H

_i32 = jnp.int32
_f32 = jnp.float32


def _make_prep():
    out_type = (
        jax.ShapeDtypeStruct((NC, NS, PPC, NBMAX, K), _i32),
        jax.ShapeDtypeStruct((NC, NS, PPC, NBMAX, K), _i32),
        jax.ShapeDtypeStruct((NC, NS, PPC, 16), _i32),
        jax.ShapeDtypeStruct((NPART, CLIVE, H), _f32),
    )

    scratch = [
        pltpu.VMEM((NBMAX, K), _i32),
        pltpu.VMEM((NBMAX, K), _i32),
        pltpu.VMEM((EB,), _i32),
        pltpu.VMEM((EB,), _i32),
        pltpu.VMEM((EB,), _i32),
        pltpu.VMEM((16,), _i32),
        pltpu.VMEM((CB_ROWS, H), _f32),
        pltpu.VMEM((CB_ROWS, H), _f32),
        pltpu.VMEM((CB_ROWS,), _i32),
        pltpu.SemaphoreType.DMA,
        pltpu.VMEM_SHARED((CB_ROWS, H), _f32),
    ]

    def body(src_hbm, dst_hbm, rel_hbm, glist, flist, nbv, c_hbm,
             gbuf, fbuf, ebuf_s, ebuf_d, ebuf_r, nstage, cb, zbuf48,
             idrows, sem, cnt_sh):
        cid = lax.axis_index("c")
        sid = lax.axis_index("s")

        zero16f = jnp.zeros((16,), _f32)
        ones16f = jnp.ones((16,), _f32)
        lane = lax.iota(_i32, 16)

        def zb(i, _):
            rr = i >> 3
            cc = (i & 7) * 16
            zbuf48[rr, pl.ds(cc, 16)] = zero16f
            return 0
        lax.fori_loop(0, CB_ROWS * 8, zb, 0)
        for j in range(CB_ROWS // 16):
            idrows[pl.ds(j * 16, 16)] = lane + j * 16

        def partition(pp, _):
            lo = (cid * PPC + pp) * NCHUNK

            def zcb(i, _):
                rr = i >> 3
                cc = (i & 7) * 16
                cb[rr, pl.ds(cc, 16)] = zero16f
                return 0
            lax.fori_loop(0, CB_ROWS * 8, zcb, 0)

            @pl.when(sid == 0)
            def _():
                pltpu.sync_copy(zbuf48, cnt_sh)
            plsc.subcore_barrier()

            def scan_block(blk, off_v):
                ebase = sid * EW + blk * EB
                c1 = pltpu.async_copy(src_hbm.at[pl.ds(ebase, EB)],
                                      ebuf_s, sem)
                c2 = pltpu.async_copy(dst_hbm.at[pl.ds(ebase, EB)],
                                      ebuf_d, sem)
                c3 = pltpu.async_copy(rel_hbm.at[pl.ds(ebase, EB)],
                                      ebuf_r, sem)
                c1.wait(); c2.wait(); c3.wait()

                def inner(i, off_v):
                    d = ebuf_d[pl.ds(i * 16, 16)]
                    s = ebuf_s[pl.ds(i * 16, 16)]
                    r = ebuf_r[pl.ds(i * 16, 16)]
                    m = (d >= lo) & (d < lo + NCHUNK)
                    si = r * NCHUNK + (d - lo)
                    srow = lax.shift_right_logical(si, 7)
                    scol = lax.bitwise_and(si, H - 1)
                    plsc.addupdate_scatter(cb, [srow, scol], ones16f,
                                           mask=m)
                    ones = jnp.where(m, 1, 0)
                    pos = off_v + plsc.cumsum(ones) - 1
                    prow = lax.shift_right_logical(pos, KSH)
                    pcol = lax.bitwise_and(pos, K - 1)
                    plsc.store_scatter(gbuf, [prow, pcol], s, mask=m)
                    plsc.store_scatter(fbuf, [prow, pcol], si, mask=m)
                    return off_v + plsc.all_reduce_population_count(m)

                return lax.fori_loop(0, EB // 16, inner, off_v)

            off_v = lax.fori_loop(0, EW // EB, scan_block,
                                  jnp.zeros((16,), _i32))
            matched = jnp.sum(jnp.where(lane == 0, off_v, 0))

            trash16 = jnp.full((16,), TRASH, _i32)
            zero16i = jnp.zeros((16,), _i32)
            def padw(j, _):
                p = matched + j * 16 + lane
                prow = lax.shift_right_logical(p, KSH)
                pcol = lax.bitwise_and(p, K - 1)
                plsc.store_scatter(fbuf, [prow, pcol], trash16)
                plsc.store_scatter(gbuf, [prow, pcol], zero16i)
                return 0
            lax.fori_loop(0, K // 16, padw, 0)

            nb = (matched + (K - 1)) // K
            nstage[pl.ds(0, 16)] = jnp.zeros((16,), _i32) + nb
            pltpu.sync_copy(nstage, nbv.at[cid, sid, pp])

            nch = (matched + K + 16 * K - 1) // (16 * K)
            def wch(j, _):
                w1 = pltpu.async_copy(
                    gbuf.at[pl.ds(j * 16, 16)],
                    glist.at[cid, sid, pp, pl.ds(j * 16, 16)], sem)
                w2 = pltpu.async_copy(
                    fbuf.at[pl.ds(j * 16, 16)],
                    flist.at[cid, sid, pp, pl.ds(j * 16, 16)], sem)
                w1.wait(); w2.wait()
                return 0
            lax.fori_loop(0, nch, wch, 0)

            pltpu.sync_copy(cb, cnt_sh.at[idrows], add=True)
            plsc.subcore_barrier()

            @pl.when(sid == 0)
            def _():
                pltpu.sync_copy(cnt_sh.at[pl.ds(0, CLIVE)],
                                c_hbm.at[cid * PPC + pp])
            plsc.subcore_barrier()
            return 0

        lax.fori_loop(0, PPC, partition, 0)

    mesh = plsc.VectorSubcoreMesh(core_axis_name="c", subcore_axis_name="s")
    return pl.kernel(body, out_type=out_type,
                     mesh=mesh, scratch_types=scratch,
                     compiler_params=pltpu.CompilerParams(
                         needs_layout_passes=False))


def _make_agg():
    out_type = jax.ShapeDtypeStruct((R, N, H), _f32)

    scratch = [
        pltpu.VMEM((NBMAX, K), _i32),
        pltpu.VMEM((NBMAX, K), _i32),
        pltpu.VMEM((16,), _i32),
        pltpu.VMEM((K, H), _f32),
        pltpu.VMEM((K, H), _f32),
        pltpu.VMEM((64, H), _f32),
        pltpu.SemaphoreType.DMA,
        pltpu.SemaphoreType.DMA,
        pltpu.SemaphoreType.DMA,
        pltpu.VMEM_SHARED((ACC_TOTAL, H), _f32),
    ]

    def body(x_hbm, glist, flist, nbv, a_hbm,
             gbuf, fbuf, nstage, rows0, rows1, zbuf,
             gsem, ssem, lsem, acc_sh):
        cid = lax.axis_index("c")
        sid = lax.axis_index("s")

        zero16f = jnp.zeros((16,), _f32)
        lane = lax.iota(_i32, 16)

        def zb(i, _):
            rr = i >> 3
            cc = (i & 7) * 16
            zbuf[rr, pl.ds(cc, 16)] = zero16f
            return 0
        lax.fori_loop(0, 64 * 8, zb, 0)

        def load_lists(pp, wait):
            pltpu.sync_copy(nbv.at[cid, sid, pp], nstage)
            nv = nstage[pl.ds(0, 16)]
            nb = jnp.sum(jnp.where(lane == 0, nv, 0))
            nch = (nb + 15) // 16
            def rch(j, _):
                r1 = pltpu.async_copy(
                    glist.at[cid, sid, pp, pl.ds(j * 16, 16)],
                    gbuf.at[pl.ds(j * 16, 16)], lsem)
                r2 = pltpu.async_copy(
                    flist.at[cid, sid, pp, pl.ds(j * 16, 16)],
                    fbuf.at[pl.ds(j * 16, 16)], lsem)
                if wait:
                    r1.wait(); r2.wait()
                return 0
            lax.fori_loop(0, nch, rch, 0)
            return nb

        def drain_lists(pp, nb):
            nch = (nb + 15) // 16
            def dch(j, _):
                pltpu.make_async_copy(
                    glist.at[cid, sid, pp, pl.ds(j * 16, 16)],
                    gbuf.at[pl.ds(j * 16, 16)], lsem).wait()
                pltpu.make_async_copy(
                    flist.at[cid, sid, pp, pl.ds(j * 16, 16)],
                    fbuf.at[pl.ds(j * 16, 16)], lsem).wait()
                return 0
            lax.fori_loop(0, nch, dch, 0)

        nb0 = load_lists(0, True)

        def partition(pp, nb):
            lo = (cid * PPC + pp) * NCHUNK

            base = pl.multiple_of(sid * ZROWS, 8)
            for j in range(ZROWS // 64):
                pltpu.sync_copy(zbuf, acc_sh.at[pl.ds(base + j * 64, 64)])
            pltpu.sync_copy(zbuf.at[pl.ds(0, ZROWS % 64)],
                            acc_sh.at[pl.ds(base + (ZROWS // 64) * 64,
                                            ZROWS % 64)])

            plsc.subcore_barrier()

            @pl.when(nb > 0)
            def _():
                pltpu.async_copy(x_hbm.at[gbuf.at[0]], rows0, gsem)

            npairs = (nb + 1) // 2
            def pair(g, _):
                b0 = 2 * g
                b1 = b0 + 1
                pltpu.make_async_copy(x_hbm.at[gbuf.at[b0]], rows0,
                                      gsem).wait()

                @pl.when(b1 < nb)
                def _():
                    pltpu.async_copy(x_hbm.at[gbuf.at[b1]], rows1, ssem)
                pltpu.sync_copy(rows0, acc_sh.at[fbuf.at[b0]], add=True)

                @pl.when(b1 < nb)
                def _():
                    pltpu.make_async_copy(x_hbm.at[gbuf.at[b1]], rows1,
                                          ssem).wait()

                    @pl.when(b1 + 1 < nb)
                    def _():
                        pltpu.async_copy(x_hbm.at[gbuf.at[b1 + 1]], rows0,
                                         gsem)
                    pltpu.sync_copy(rows1, acc_sh.at[fbuf.at[b1]],
                                    add=True)
                return 0
            lax.fori_loop(0, npairs, pair, 0)

            plsc.subcore_barrier()

            nb2 = lax.cond(pp + 1 < PPC,
                           lambda: load_lists(pp + 1, False),
                           lambda: _i32(0))

            r = sid % R
            half = sid // R
            row0 = pl.multiple_of(r * NCHUNK + half * WHALF, 8)
            node0 = pl.multiple_of(lo + half * WHALF, 8)

            @pl.when(node0 + WHALF <= N)
            def _():
                pltpu.sync_copy(acc_sh.at[pl.ds(row0, WHALF)],
                                a_hbm.at[r, pl.ds(node0, WHALF)])

            tail = N % WHALF

            @pl.when((node0 + WHALF > N) & (node0 < N))
            def _():
                pltpu.sync_copy(acc_sh.at[pl.ds(row0, tail)],
                                a_hbm.at[r, pl.ds(node0, tail)])
            plsc.subcore_barrier()

            @pl.when(pp + 1 < PPC)
            def _():
                drain_lists(pp + 1, nb2)
            return nb2

        lax.fori_loop(0, PPC, partition, nb0)

    mesh = plsc.VectorSubcoreMesh(core_axis_name="c", subcore_axis_name="s")
    return pl.kernel(body, out_type=out_type,
                     mesh=mesh, scratch_types=scratch,
                     compiler_params=pltpu.CompilerParams(
                         needs_layout_passes=False))


_prep = _make_prep()
_agg = _make_agg()


_BM = 512
_BOUNDS = (0, 4000, 7000, 9000, 9500, 10000)


def _proj_body(x_ref, w_ref, b_ref, o_ref):
    m = pl.program_id(0)
    row = m * _BM + lax.broadcasted_iota(_i32, (_BM, 1), 0)
    x = x_ref[...]
    acc = jnp.zeros((_BM, H), _f32)
    for t in range(5):
        y = jnp.dot(x, w_ref[t], preferred_element_type=_f32) + b_ref[t]
        sel = (row >= _BOUNDS[t]) & (row < _BOUNDS[t + 1])
        acc = jnp.where(sel, y, acc)
    o_ref[...] = acc


def _proj(x_cat, w_stack, b_stack):
    grid = (pl.cdiv(N, _BM),)
    return pl.pallas_call(
        _proj_body,
        grid=grid,
        in_specs=[
            pl.BlockSpec((_BM, H), lambda m: (m, 0)),
            pl.BlockSpec((5, H, H), lambda m: (0, 0, 0)),
            pl.BlockSpec((5, H), lambda m: (0, 0)),
        ],
        out_specs=pl.BlockSpec((_BM, H), lambda m: (m, 0)),
        out_shape=jax.ShapeDtypeStruct((N, H), _f32),
    )(x_cat, w_stack, b_stack)


def _make_conv_tc(relu):
    def body(x_ref, a_ref, c_ref, wroot_ref, wrel_ref, b_ref, o_ref):
        acc = jnp.dot(x_ref[...], wroot_ref[...], preferred_element_type=_f32)
        for r in range(R):
            cnt = c_ref[r][:, None]
            rec = 1.0 / jnp.maximum(cnt, 1.0)
            acc = acc + jnp.dot(a_ref[r] * rec, wrel_ref[r],
                                preferred_element_type=_f32)
        acc = acc + b_ref[...]
        if relu:
            acc = jnp.maximum(acc, 0.0)
        o_ref[...] = acc

    def run(x, a, cnt, w_root, w_rel, b):
        grid = (pl.cdiv(N, _BM),)
        return pl.pallas_call(
            body,
            grid=grid,
            in_specs=[
                pl.BlockSpec((_BM, H), lambda m: (m, 0)),
                pl.BlockSpec((R, _BM, H), lambda m: (0, m, 0)),
                pl.BlockSpec((R, _BM), lambda m: (0, m)),
                pl.BlockSpec((H, H), lambda m: (0, 0)),
                pl.BlockSpec((R, H, H), lambda m: (0, 0, 0)),
                pl.BlockSpec((H,), lambda m: (0,)),
            ],
            out_specs=pl.BlockSpec((_BM, H), lambda m: (m, 0)),
            out_shape=jax.ShapeDtypeStruct((N, H), _f32),
        )(x, a, cnt, w_root, w_rel, b)

    return run


_conv_relu = _make_conv_tc(True)
_conv_plain = _make_conv_tc(False)


def kernel(x_user, x_food, x_ingredient, x_category, x_habit, edge_index,
           edge_type, W_user, b_user, W_food, b_food, W_ing, b_ing, W_cat,
           b_cat, W_hab, b_hab, W_rel1, W_root1, b1, W_rel2, W_root2, b2):
    x_cat = jnp.concatenate([x_user, x_food, x_ingredient, x_category,
                             x_habit], axis=0).astype(_f32)
    w_stack = jnp.stack([W_user, W_food, W_ing, W_cat, W_hab])
    b_stack = jnp.stack([b_user, b_food, b_ing, b_cat, b_hab])
    x_all = _proj(x_cat, w_stack, b_stack)

    src = edge_index[0].astype(_i32)
    dst = edge_index[1].astype(_i32)
    et = edge_type.astype(_i32)

    glist, flist, nbv, craw = _prep(src, dst, et)
    cnt = (craw.reshape(NPART, R, NCHUNK)
               .transpose(1, 0, 2).reshape(R, NPART * NCHUNK)[:, :N])

    a1 = _agg(x_all, glist, flist, nbv)
    h = _conv_relu(x_all, a1, cnt, W_root1, W_rel1, b1)
    a2 = _agg(h, glist, flist, nbv)
    out = _conv_plain(h, a2, cnt, W_root2, W_rel2, b2)
    return out
